# baseline (speedup 1.0000x reference)
import functools

import jax
import jax.numpy as jnp
from jax import lax
from jax.experimental import pallas as pl
from jax.experimental.pallas import tpu as pltpu
from jax.experimental.pallas import tpu_sc as plsc

HIDDEN = 128
N_EDGES = 320000
N_X = 10000

SC_WORKERS = 32
SC_CHUNK = 200
SC_TRIPS = 15
SC_ROWS_W = SC_CHUNK * SC_TRIPS
SC_ROWS = SC_ROWS_W * SC_WORKERS
TC_ROWS = N_EDGES - SC_ROWS

TC_GRID = 50
BE = TC_ROWS // TC_GRID
BX = N_X // TC_GRID


def _sc_sum_body(e_hbm, out_hbm, buf0, buf1, acc_v, sem0, sem1):
    cid = lax.axis_index("c")
    sid = lax.axis_index("s")
    wid = sid * 2 + cid
    base = TC_ROWS + wid * SC_ROWS_W

    bufs = (buf0, buf1)
    sems = (sem0, sem1)
    copies = [None] * SC_TRIPS
    copies[0] = pltpu.async_copy(e_hbm.at[pl.ds(base, SC_CHUNK)], bufs[0], sems[0])

    carry = tuple(jnp.zeros((16,), jnp.float32) for _ in range(8))

    def row_add(buf):
        def body(r, cr):
            return tuple(cr[c] + buf[r, pl.ds(c * 16, 16)] for c in range(8))
        return body

    for t in range(SC_TRIPS):
        if t + 1 < SC_TRIPS:
            copies[t + 1] = pltpu.async_copy(
                e_hbm.at[pl.ds(base + (t + 1) * SC_CHUNK, SC_CHUNK)],
                bufs[(t + 1) % 2], sems[(t + 1) % 2])
        copies[t].wait()
        carry = lax.fori_loop(0, SC_CHUNK, row_add(bufs[t % 2]), carry)

    zero16 = jnp.zeros((16,), jnp.float32)
    for rr in range(8):
        for c in range(8):
            acc_v[rr, pl.ds(c * 16, 16)] = carry[c] if rr == 0 else zero16
    pltpu.sync_copy(acc_v, out_hbm.at[pl.ds(wid * 8, 8)])


@functools.partial(jax.jit)
def _sc_sum(edges):
    k = functools.partial(
        pl.kernel,
        out_type=jax.ShapeDtypeStruct((SC_WORKERS * 8, HIDDEN), jnp.float32),
        mesh=plsc.VectorSubcoreMesh(core_axis_name="c", subcore_axis_name="s"),
        scratch_types=[
            pltpu.VMEM((SC_CHUNK, HIDDEN), jnp.float32),
            pltpu.VMEM((SC_CHUNK, HIDDEN), jnp.float32),
            pltpu.VMEM((8, HIDDEN), jnp.float32),
            pltpu.SemaphoreType.DMA,
            pltpu.SemaphoreType.DMA,
        ],
    )(_sc_sum_body)
    return k(edges)


def _tree_sum8(a):
    rows = a.shape[0]
    if rows > 128 and rows % 128 == 0:
        a = a.reshape(rows // 128, 128, HIDDEN).sum(axis=0)
        rows = 128
    while rows > 8 and rows % 16 == 0:
        rows //= 2
        a = a[:rows] + a[rows:]
    if rows > 8:
        a = a.reshape(rows // 8, 8, HIDDEN).sum(axis=0)
    return a


def _tc_sum_kernel(x_ref, e_ref, out_ref, acc_ref):
    i = pl.program_id(0)

    @pl.when(i == 0)
    def _init():
        acc_ref[...] = jnp.zeros_like(acc_ref)

    acc_ref[0:8, :] += _tree_sum8(x_ref[...])
    acc_ref[8:16, :] += _tree_sum8(e_ref[...])

    @pl.when(i == TC_GRID - 1)
    def _finish():
        out_ref[...] = acc_ref[...]


def _tc_sum(x, edges):
    return pl.pallas_call(
        _tc_sum_kernel,
        grid=(TC_GRID,),
        in_specs=[
            pl.BlockSpec((BX, HIDDEN), lambda i: (i, 0)),
            pl.BlockSpec((BE, HIDDEN), lambda i: (i, 0)),
        ],
        out_specs=pl.BlockSpec((16, HIDDEN), lambda i: (0, 0)),
        out_shape=jax.ShapeDtypeStruct((16, HIDDEN), jnp.float32),
        scratch_shapes=[pltpu.VMEM((16, HIDDEN), jnp.float32)],
        compiler_params=pltpu.CompilerParams(
            dimension_semantics=("arbitrary",),
        ),
    )(x, edges)


def _mlp_kernel(tcp_ref, scp_ref, g_ref, w1_ref, b1_ref, w2_ref, b2_ref,
                gamma_ref, beta_ref, out_ref):
    sn = jnp.sum(tcp_ref[0:8, :], axis=0, keepdims=True)
    e8 = tcp_ref[8:16, :] + _tree_sum8(scp_ref[...])
    se = jnp.sum(e8, axis=0, keepdims=True)
    g = g_ref[...]
    h = (jnp.dot(sn, w1_ref[0:HIDDEN, :], preferred_element_type=jnp.float32)
         + jnp.dot(se, w1_ref[HIDDEN:2 * HIDDEN, :], preferred_element_type=jnp.float32)
         + jnp.dot(g, w1_ref[2 * HIDDEN:3 * HIDDEN, :], preferred_element_type=jnp.float32)
         + b1_ref[...])
    h = jnp.maximum(h, 0.0)
    out = jnp.dot(h, w2_ref[...], preferred_element_type=jnp.float32) + b2_ref[...]
    mean = jnp.mean(out, axis=-1, keepdims=True)
    var = jnp.mean((out - mean) ** 2, axis=-1, keepdims=True)
    out_ref[...] = ((out - mean) * jax.lax.rsqrt(var + 1e-5)
                    * gamma_ref[...] + beta_ref[...])


def _mlp(tc_part, sc_part, global_attr, W1, b1r, W2, b2r, gammar, betar):
    return pl.pallas_call(
        _mlp_kernel,
        out_shape=jax.ShapeDtypeStruct((1, HIDDEN), jnp.float32),
    )(tc_part, sc_part, global_attr, W1, b1r, W2, b2r, gammar, betar)


def kernel(x, edge_attr_updated, global_attr, W1, b1, W2, b2, gamma, beta):
    tc_part = _tc_sum(x, edge_attr_updated)
    sc_part = _sc_sum(edge_attr_updated)
    return _mlp(tc_part, sc_part, global_attr, W1,
                b1.reshape(1, HIDDEN), W2, b2.reshape(1, HIDDEN),
                gamma.reshape(1, HIDDEN), beta.reshape(1, HIDDEN))


# speedup vs baseline: 1.0018x; 1.0018x over previous
import functools

import jax
import jax.numpy as jnp
from jax import lax
from jax.experimental import pallas as pl
from jax.experimental.pallas import tpu as pltpu
from jax.experimental.pallas import tpu_sc as plsc

HIDDEN = 128
N_EDGES = 320000
N_X = 10000

SC_WORKERS = 32
SC_CHUNK = 200
SC_TRIPS = 15
SC_ROWS_W = SC_CHUNK * SC_TRIPS
SC_ROWS = SC_ROWS_W * SC_WORKERS
TC_ROWS = N_EDGES - SC_ROWS

TC_GRID = 50
BE = TC_ROWS // TC_GRID
BX = N_X // TC_GRID


def _sc_sum_body(e_hbm, out_hbm, buf0, buf1, acc_v, sem0, sem1):
    cid = lax.axis_index("c")
    sid = lax.axis_index("s")
    wid = sid * 2 + cid
    base = TC_ROWS + wid * SC_ROWS_W

    bufs = (buf0, buf1)
    sems = (sem0, sem1)
    copies = [None] * SC_TRIPS
    copies[0] = pltpu.async_copy(e_hbm.at[pl.ds(base, SC_CHUNK)], bufs[0], sems[0])

    carry = tuple(jnp.zeros((16,), jnp.float32) for _ in range(8))

    def row_add(buf):
        def body(r, cr):
            return tuple(cr[c] + buf[r, pl.ds(c * 16, 16)] for c in range(8))
        return body

    for t in range(SC_TRIPS):
        if t + 1 < SC_TRIPS:
            copies[t + 1] = pltpu.async_copy(
                e_hbm.at[pl.ds(base + (t + 1) * SC_CHUNK, SC_CHUNK)],
                bufs[(t + 1) % 2], sems[(t + 1) % 2])
        copies[t].wait()
        carry = lax.fori_loop(0, SC_CHUNK, row_add(bufs[t % 2]), carry)

    zero16 = jnp.zeros((16,), jnp.float32)
    for rr in range(8):
        for c in range(8):
            acc_v[rr, pl.ds(c * 16, 16)] = carry[c] if rr == 0 else zero16
    pltpu.sync_copy(acc_v, out_hbm.at[pl.ds(wid * 8, 8)])


@functools.partial(jax.jit)
def _sc_sum(edges):
    k = functools.partial(
        pl.kernel,
        out_type=jax.ShapeDtypeStruct((SC_WORKERS * 8, HIDDEN), jnp.float32),
        mesh=plsc.VectorSubcoreMesh(core_axis_name="c", subcore_axis_name="s"),
        scratch_types=[
            pltpu.VMEM((SC_CHUNK, HIDDEN), jnp.float32),
            pltpu.VMEM((SC_CHUNK, HIDDEN), jnp.float32),
            pltpu.VMEM((8, HIDDEN), jnp.float32),
            pltpu.SemaphoreType.DMA,
            pltpu.SemaphoreType.DMA,
        ],
    )(_sc_sum_body)
    return k(edges)


def _tree_sum8(a):
    rows = a.shape[0]
    if rows > 128 and rows % 128 == 0:
        a = a.reshape(rows // 128, 128, HIDDEN).sum(axis=0)
        rows = 128
    while rows > 8 and rows % 16 == 0:
        rows //= 2
        a = a[:rows] + a[rows:]
    if rows > 8:
        a = a.reshape(rows // 8, 8, HIDDEN).sum(axis=0)
    return a


def _tc_sum_kernel(x_ref, e_ref, out_ref, acc_ref):
    i = pl.program_id(0)

    @pl.when(i == 0)
    def _init():
        acc_ref[...] = jnp.zeros_like(acc_ref)

    acc_ref[0:8, :] += _tree_sum8(x_ref[...])
    acc_ref[8:16, :] += _tree_sum8(e_ref[...])

    @pl.when(i == TC_GRID - 1)
    def _finish():
        out_ref[...] = acc_ref[...]


def _tc_sum(x, edges):
    return pl.pallas_call(
        _tc_sum_kernel,
        grid=(TC_GRID,),
        in_specs=[
            pl.BlockSpec((BX, HIDDEN), lambda i: (i, 0)),
            pl.BlockSpec((BE, HIDDEN), lambda i: (i, 0)),
        ],
        out_specs=pl.BlockSpec((16, HIDDEN), lambda i: (0, 0)),
        out_shape=jax.ShapeDtypeStruct((16, HIDDEN), jnp.float32),
        scratch_shapes=[pltpu.VMEM((16, HIDDEN), jnp.float32)],
        compiler_params=pltpu.CompilerParams(
            dimension_semantics=("arbitrary",),
        ),
        cost_estimate=pl.CostEstimate(
            flops=(TC_ROWS + N_X) * HIDDEN,
            bytes_accessed=(TC_ROWS + N_X) * HIDDEN * 4,
            transcendentals=0,
        ),
    )(x, edges)


def _mlp_kernel(tcp_ref, scp_ref, g_ref, w1_ref, b1_ref, w2_ref, b2_ref,
                gamma_ref, beta_ref, out_ref):
    sn = jnp.sum(tcp_ref[0:8, :], axis=0, keepdims=True)
    e8 = tcp_ref[8:16, :] + _tree_sum8(scp_ref[...])
    se = jnp.sum(e8, axis=0, keepdims=True)
    g = g_ref[...]
    h = (jnp.dot(sn, w1_ref[0:HIDDEN, :], preferred_element_type=jnp.float32)
         + jnp.dot(se, w1_ref[HIDDEN:2 * HIDDEN, :], preferred_element_type=jnp.float32)
         + jnp.dot(g, w1_ref[2 * HIDDEN:3 * HIDDEN, :], preferred_element_type=jnp.float32)
         + b1_ref[...])
    h = jnp.maximum(h, 0.0)
    out = jnp.dot(h, w2_ref[...], preferred_element_type=jnp.float32) + b2_ref[...]
    mean = jnp.mean(out, axis=-1, keepdims=True)
    var = jnp.mean((out - mean) ** 2, axis=-1, keepdims=True)
    out_ref[...] = ((out - mean) * jax.lax.rsqrt(var + 1e-5)
                    * gamma_ref[...] + beta_ref[...])


def _mlp(tc_part, sc_part, global_attr, W1, b1r, W2, b2r, gammar, betar):
    return pl.pallas_call(
        _mlp_kernel,
        out_shape=jax.ShapeDtypeStruct((1, HIDDEN), jnp.float32),
    )(tc_part, sc_part, global_attr, W1, b1r, W2, b2r, gammar, betar)


def kernel(x, edge_attr_updated, global_attr, W1, b1, W2, b2, gamma, beta):
    tc_part = _tc_sum(x, edge_attr_updated)
    sc_part = _sc_sum(edge_attr_updated)
    return _mlp(tc_part, sc_part, global_attr, W1,
                b1.reshape(1, HIDDEN), W2, b2.reshape(1, HIDDEN),
                gamma.reshape(1, HIDDEN), beta.reshape(1, HIDDEN))


# speedup vs baseline: 1.5061x; 1.5034x over previous
import jax
import jax.numpy as jnp
from jax.experimental import pallas as pl
from jax.experimental.pallas import tpu as pltpu

HIDDEN = 128
N_EDGES = 320000
N_X = 10000
GRID = 25
HALF_BLOCKS = GRID
BE = N_EDGES // (2 * GRID)
BX = N_X // GRID


def _tree_sum8(a):
    rows = a.shape[0]
    if rows > 128 and rows % 128 == 0:
        a = a.reshape(rows // 128, 128, HIDDEN).sum(axis=0)
        rows = 128
    while rows > 8 and rows % 16 == 0:
        rows //= 2
        a = a[:rows] + a[rows:]
    if rows > 8:
        a = a.reshape(rows // 8, 8, HIDDEN).sum(axis=0)
    return a


def _gb_kernel(x_ref, ea_ref, eb_ref, g_ref, w1_ref, b1_ref, w2_ref, b2_ref,
               gamma_ref, beta_ref, out_ref, acc_ref):
    i = pl.program_id(0)

    @pl.when(i == 0)
    def _init():
        acc_ref[...] = jnp.zeros_like(acc_ref)

    acc_ref[0:8, :] += _tree_sum8(x_ref[...])
    acc_ref[8:16, :] += _tree_sum8(ea_ref[...])
    acc_ref[16:24, :] += _tree_sum8(eb_ref[...])

    @pl.when(i == GRID - 1)
    def _finish():
        sn = jnp.sum(acc_ref[0:8, :], axis=0, keepdims=True)
        se = jnp.sum(acc_ref[8:16, :] + acc_ref[16:24, :], axis=0, keepdims=True)
        g = g_ref[...]
        h = (jnp.dot(sn, w1_ref[0:HIDDEN, :], preferred_element_type=jnp.float32)
             + jnp.dot(se, w1_ref[HIDDEN:2 * HIDDEN, :], preferred_element_type=jnp.float32)
             + jnp.dot(g, w1_ref[2 * HIDDEN:3 * HIDDEN, :], preferred_element_type=jnp.float32)
             + b1_ref[...])
        h = jnp.maximum(h, 0.0)
        out = jnp.dot(h, w2_ref[...], preferred_element_type=jnp.float32) + b2_ref[...]
        mean = jnp.mean(out, axis=-1, keepdims=True)
        var = jnp.mean((out - mean) ** 2, axis=-1, keepdims=True)
        out_ref[...] = ((out - mean) * jax.lax.rsqrt(var + 1e-5)
                        * gamma_ref[...] + beta_ref[...])


def kernel(x, edge_attr_updated, global_attr, W1, b1, W2, b2, gamma, beta):
    b1r = b1.reshape(1, HIDDEN)
    b2r = b2.reshape(1, HIDDEN)
    gammar = gamma.reshape(1, HIDDEN)
    betar = beta.reshape(1, HIDDEN)

    const = lambda i: (0, 0)
    return pl.pallas_call(
        _gb_kernel,
        grid=(GRID,),
        in_specs=[
            pl.BlockSpec((BX, HIDDEN), lambda i: (i, 0)),
            pl.BlockSpec((BE, HIDDEN), lambda i: (i, 0)),
            pl.BlockSpec((BE, HIDDEN), lambda i: (i + HALF_BLOCKS, 0)),
            pl.BlockSpec((1, HIDDEN), const),
            pl.BlockSpec((3 * HIDDEN, HIDDEN), const),
            pl.BlockSpec((1, HIDDEN), const),
            pl.BlockSpec((HIDDEN, HIDDEN), const),
            pl.BlockSpec((1, HIDDEN), const),
            pl.BlockSpec((1, HIDDEN), const),
            pl.BlockSpec((1, HIDDEN), const),
        ],
        out_specs=pl.BlockSpec((1, HIDDEN), const),
        out_shape=jax.ShapeDtypeStruct((1, HIDDEN), jnp.float32),
        scratch_shapes=[pltpu.VMEM((24, HIDDEN), jnp.float32)],
        compiler_params=pltpu.CompilerParams(
            dimension_semantics=("arbitrary",),
        ),
    )(x, edge_attr_updated, edge_attr_updated, global_attr, W1, b1r, W2,
      b2r, gammar, betar)
